# baseline (speedup 1.0000x reference)
import jax
import jax.numpy as jnp
from jax import lax
from jax.experimental import pallas as pl
from jax.experimental.pallas import tpu as pltpu

N = 4096
D = 512
K = 512
BM = 512
NBUF = 4096
NEG_INF = float("-inf")


def _p1(ref_ref, src_ref, d_ref):
    d_ref[...] = lax.dot_general(
        ref_ref[...], src_ref[...],
        dimension_numbers=(((1,), (1,)), ((), ())),
        preferred_element_type=jnp.float32)


def _p2(n_ref, rmax_ref, thr_ref, rm_acc):
    i = pl.program_id(0)
    rm = jnp.max(n_ref[...], axis=1)
    rmax_ref[...] = rm
    rm_acc[pl.ds(i * BM, BM)] = rm

    @pl.when(i == pl.num_programs(0) - 1)
    def _():
        bits = lax.bitcast_convert_type(rm_acc[...], jnp.int32)

        def body(_, carry):
            lo, hi = carry
            mid = lo + (hi - lo) // 2
            cnt = jnp.sum((bits >= mid).astype(jnp.int32))
            ge = cnt >= K
            return (jnp.where(ge, mid, lo), jnp.where(ge, hi, mid))

        lo, hi = lax.fori_loop(0, 31, body, (jnp.int32(0), jnp.int32(0x7F800000)))
        thr_ref[...] = lax.bitcast_convert_type(jnp.full((1024,), lo), jnp.float32)


def _p4(gv_ref, gi_ref, sc_ref, fi_ref, ranks):
    gv = gv_ref[...]
    gi = gi_ref[...]

    def rank_chunk(c, _):
        vi = gv_ref[pl.ds(128 * c, 128)]
        ii = gi_ref[pl.ds(128 * c, 128)]
        gt = (gv[None, :] > vi[:, None])
        tie = (gv[None, :] == vi[:, None]) & (gi[None, :] < ii[:, None])
        r = jnp.sum((gt | tie).astype(jnp.int32), axis=1)
        ranks[pl.ds(128 * c, 128)] = r
        return 0

    lax.fori_loop(0, NBUF // 128, rank_chunk, 0, unroll=False)

    rk = ranks[...]
    for c in range(K // 128):
        kvec = lax.iota(jnp.int32, 128)[:, None] + 128 * c
        m = rk[None, :] == kvec
        sc_ref[pl.ds(128 * c, 128)] = jnp.sum(
            jnp.where(m, gv[None, :], 0.0), axis=1)
        fi_ref[pl.ds(128 * c, 128)] = jnp.sum(
            jnp.where(m, gi[None, :], 0), axis=1)


def kernel(ref_feats, src_feats, ref_masks, src_masks):
    n_ref_pts = ref_masks.shape[0]
    n_src_pts = src_masks.shape[0]
    ref_indices = jnp.nonzero(ref_masks, size=n_ref_pts, fill_value=0)[0]
    src_indices = jnp.nonzero(src_masks, size=n_src_pts, fill_value=0)[0]

    d = pl.pallas_call(
        _p1,
        grid=(N // BM,),
        in_specs=[
            pl.BlockSpec((BM, D), lambda i: (i, 0)),
            pl.BlockSpec((N, D), lambda i: (0, 0)),
        ],
        out_specs=pl.BlockSpec((BM, N), lambda i: (i, 0)),
        out_shape=jax.ShapeDtypeStruct((N, N), jnp.float32),
    )(ref_feats, src_feats)

    s = jnp.exp(-(2.0 - 2.0 * d))
    rsum = jnp.sum(s, axis=1, keepdims=True)
    csum = jnp.sum(s, axis=0, keepdims=True)
    n = (s / rsum) * (s / csum)

    rmax, thr = pl.pallas_call(
        _p2,
        grid=(N // BM,),
        in_specs=[pl.BlockSpec((BM, N), lambda i: (i, 0))],
        out_specs=[
            pl.BlockSpec((BM,), lambda i: (i,)),
            pl.BlockSpec((1024,), lambda i: (0,)),
        ],
        out_shape=[
            jax.ShapeDtypeStruct((N,), jnp.float32),
            jax.ShapeDtypeStruct((1024,), jnp.float32),
        ],
        scratch_shapes=[pltpu.VMEM((N,), jnp.float32)],
    )(n)

    T = thr[0]
    flat = n.reshape(-1)
    gidx = jnp.nonzero(flat >= T, size=NBUF, fill_value=N * N)[0].astype(jnp.int32)
    pad = gidx == N * N
    gvals = jnp.where(pad, NEG_INF,
                      jnp.take(flat, jnp.minimum(gidx, N * N - 1)))

    corr_scores, flat_idx = pl.pallas_call(
        _p4,
        out_shape=[
            jax.ShapeDtypeStruct((K,), jnp.float32),
            jax.ShapeDtypeStruct((K,), jnp.int32),
        ],
        scratch_shapes=[pltpu.VMEM((NBUF,), jnp.int32)],
    )(gvals, gidx)

    ref_sel = flat_idx // N
    src_sel = flat_idx % N
    ref_corr = jnp.take(ref_indices, ref_sel, axis=0)
    src_corr = jnp.take(src_indices, src_sel, axis=0)
    return (ref_corr, src_corr, corr_scores)
